# baseline (speedup 1.0000x reference)
import functools

import jax
import jax.numpy as jnp
from jax import lax
from jax.experimental import pallas as pl
from jax.experimental.pallas import tpu as pltpu
from jax.experimental.pallas import tpu_sc as plsc

N = 4096
K = 20
B = 8
NK = N * K
RB = 128
NB = N // RB
DPAD = 128
NSUB = 32
EPW = NK // NSUB
G = EPW // 128

_NOISE01_CACHE = []


def _noise01():
    if not _NOISE01_CACHE:
        _NOISE01_CACHE.append(
            jax.random.uniform(jax.random.key(42), (N, N), dtype=jnp.float32)
            * jnp.float32(0.01))
    return _NOISE01_CACHE[0]


def _tc_body(nv1_ref, nv2_ref, noise_ref,
             posa_ref, isreal_ref, rv_ref, cv_ref, val_ref, nnz_ref,
             ee_ref, carry_ref):
    b = pl.program_id(0)

    @pl.when(b == 0)
    def _init():
        ee_ref[...] = jnp.tanh(2.0 * nv2_ref[...])
        carry_ref[0] = jnp.int32(0)

    de = jnp.tanh(2.0 * nv1_ref[...])
    dot = lax.dot_general(de, ee_ref[...],
                          dimension_numbers=(((1,), (1,)), ((), ())),
                          preferred_element_type=jnp.float32)
    adj = jnp.maximum(jnp.tanh(2.0 * dot), 0.0)
    scores = adj + noise_ref[...]
    col = lax.broadcasted_iota(jnp.int32, (RB, N), 1)
    big = jnp.int32(1 << 30)
    idx_cols = []
    val_cols = []
    for _ in range(K):
        m = jnp.max(scores, axis=1, keepdims=True)
        cand = jnp.where(scores == m, col, big)
        idx_t = jnp.min(cand, axis=1, keepdims=True)
        sel = col == idx_t
        val_t = jnp.sum(jnp.where(sel, adj, 0.0), axis=1, keepdims=True)
        scores = jnp.where(sel, -1.0, scores)
        idx_cols.append(idx_t)
        val_cols.append(val_t)
    idx20 = jnp.concatenate(idx_cols, axis=1)
    val20 = jnp.concatenate(val_cols, axis=1)
    real = val20 > 0.0
    kio = lax.broadcasted_iota(jnp.int32, (RB, K), 1)
    key = jnp.where(real, idx20, N + kio)
    rank = jnp.zeros((RB, K), jnp.int32)
    for j in range(K):
        rank = rank + jnp.where(key[:, j:j + 1] < key, 1, 0)
    cnt = jnp.sum(jnp.where(real, 1, 0), axis=1, keepdims=True)
    rio = lax.broadcasted_iota(jnp.int32, (RB, RB), 0)
    cio = lax.broadcasted_iota(jnp.int32, (RB, RB), 1)
    tril = jnp.where(cio < rio, 1.0, 0.0)
    cstart = lax.dot_general(tril, cnt.astype(jnp.float32),
                             dimension_numbers=(((1,), (0,)), ((), ())),
                             preferred_element_type=jnp.float32)
    carry = carry_ref[0]
    row_start = carry + cstart.astype(jnp.int32)
    carry_ref[0] = carry + jnp.sum(cnt)
    rglob = RB * b + lax.broadcasted_iota(jnp.int32, (RB, 1), 0)
    padstart = K * rglob - row_start
    posa_ref[...] = jnp.where(real, row_start + rank, padstart + rank - cnt)
    isreal_ref[...] = jnp.where(real, 1, 0)
    rv_ref[...] = jnp.where(real, rglob, 0)
    cv_ref[...] = jnp.where(real, idx20, 0)
    val_ref[...] = val20
    nnz_ref[...] = jnp.full((8, 128), carry_ref[0], jnp.int32)


_tc_call = pl.pallas_call(
    _tc_body,
    grid=(NB,),
    in_specs=[
        pl.BlockSpec((RB, DPAD), lambda b: (b, 0)),
        pl.BlockSpec((N, DPAD), lambda b: (0, 0)),
        pl.BlockSpec((RB, N), lambda b: (b, 0)),
    ],
    out_specs=[
        pl.BlockSpec((RB, K), lambda b: (b, 0)),
        pl.BlockSpec((RB, K), lambda b: (b, 0)),
        pl.BlockSpec((RB, K), lambda b: (b, 0)),
        pl.BlockSpec((RB, K), lambda b: (b, 0)),
        pl.BlockSpec((RB, K), lambda b: (b, 0)),
        pl.BlockSpec((8, 128), lambda b: (0, 0)),
    ],
    out_shape=[
        jax.ShapeDtypeStruct((N, K), jnp.int32),
        jax.ShapeDtypeStruct((N, K), jnp.int32),
        jax.ShapeDtypeStruct((N, K), jnp.int32),
        jax.ShapeDtypeStruct((N, K), jnp.int32),
        jax.ShapeDtypeStruct((N, K), jnp.float32),
        jax.ShapeDtypeStruct((8, 128), jnp.int32),
    ],
    scratch_shapes=[
        pltpu.VMEM((N, DPAD), jnp.float32),
        pltpu.SMEM((1,), jnp.int32),
    ],
)


def _sc_body(posa_hbm, isreal_hbm, rv_hbm, cv_hbm, val_hbm, nnz_hbm,
             er_hbm, ec_hbm, hew_hbm,
             posa_v, isreal_v, rv_v, cv_v, val_v, nnz_v,
             idxe_v, idxh_v, er_st, ec_st, sem):
    wid = lax.axis_index("s") * 2 + lax.axis_index("c")
    base = wid * EPW
    with jax.named_scope("sc_load"):
        pltpu.sync_copy(posa_hbm.at[pl.ds(base, EPW)], posa_v)
        pltpu.sync_copy(isreal_hbm.at[pl.ds(base, EPW)], isreal_v)
        pltpu.sync_copy(rv_hbm.at[pl.ds(base, EPW)], rv_v)
        pltpu.sync_copy(cv_hbm.at[pl.ds(base, EPW)], cv_v)
        pltpu.sync_copy(val_hbm.at[pl.ds(base, EPW)], val_v)
        pltpu.sync_copy(nnz_hbm.at[pl.ds(0, 16)], nnz_v)
    nnz = nnz_v[...]

    def fill_batch(i, _):
        def fill_body(g, _):
            for v in range(8):
                off = g * 128 + v * 16
                soff = i * EPW + off
                pos = posa_v[pl.ds(off, 16)]
                isr = isreal_v[pl.ds(off, 16)]
                pad = 1 - isr
                idxe_v[i * G + g, pl.ds(v * 16, 16)] = pos + pad * nnz + i * NK
                stride = isr * nnz + pad * (NK - nnz)
                idxh_v[i * G + g, pl.ds(v * 16, 16)] = (
                    pos + pad * (8 * nnz) + i * stride)
                er_st[pl.ds(soff, 16)] = rv_v[pl.ds(off, 16)] + i * N
                ec_st[pl.ds(soff, 16)] = cv_v[pl.ds(off, 16)] + i * N
            return 0

        lax.fori_loop(0, G, fill_body, 0)
        return 0

    with jax.named_scope("sc_fill"):
        lax.fori_loop(0, B, fill_batch, 0)

    def batch_body(i, _):
        def scat_body(g, _):
            pltpu.async_copy(er_st.at[pl.ds(i * EPW + g * 128, 128)],
                             er_hbm.at[idxe_v.at[i * G + g]], sem)
            pltpu.async_copy(ec_st.at[pl.ds(i * EPW + g * 128, 128)],
                             ec_hbm.at[idxe_v.at[i * G + g]], sem)
            pltpu.async_copy(val_v.at[pl.ds(g * 128, 128)],
                             hew_hbm.at[idxh_v.at[i * G + g]], sem)
            return 0

        lax.fori_loop(0, G, scat_body, 0)
        pltpu.make_async_copy(posa_hbm.at[pl.ds(0, EPW)],
                              er_st.at[pl.ds(i * EPW, EPW)], sem).wait()
        pltpu.make_async_copy(posa_hbm.at[pl.ds(0, EPW)],
                              ec_st.at[pl.ds(i * EPW, EPW)], sem).wait()
        pltpu.make_async_copy(val_hbm.at[pl.ds(0, EPW)], val_v, sem).wait()
        return 0

    with jax.named_scope("sc_scatter"):
        lax.fori_loop(0, B, batch_body, 0)


_SC_CALL_CACHE = []


def _sc_call_build():
    return functools.partial(
        pl.kernel,
        mesh=plsc.VectorSubcoreMesh(core_axis_name="c", subcore_axis_name="s"),
        out_type=[
            jax.ShapeDtypeStruct((B * NK,), jnp.int32),
            jax.ShapeDtypeStruct((B * NK,), jnp.int32),
            jax.ShapeDtypeStruct((B * NK,), jnp.float32),
        ],
        scratch_types=[
            pltpu.VMEM((EPW,), jnp.int32),
            pltpu.VMEM((EPW,), jnp.int32),
            pltpu.VMEM((EPW,), jnp.int32),
            pltpu.VMEM((EPW,), jnp.int32),
            pltpu.VMEM((EPW,), jnp.float32),
            pltpu.VMEM((16,), jnp.int32),
            pltpu.VMEM((B * G, 128), jnp.int32),
            pltpu.VMEM((B * G, 128), jnp.int32),
            pltpu.VMEM((B * EPW,), jnp.int32),
            pltpu.VMEM((B * EPW,), jnp.int32),
            pltpu.SemaphoreType.DMA,
        ],
    )(_sc_body)


def kernel(x, nodevec1, nodevec2):
    del x
    nv1 = jnp.pad(nodevec1, ((0, 0), (0, DPAD - nodevec1.shape[1])))
    nv2 = jnp.pad(nodevec2, ((0, 0), (0, DPAD - nodevec2.shape[1])))
    posa, isreal, rv, cv, val, nnz = _tc_call(nv1, nv2, _noise01())
    if not _SC_CALL_CACHE:
        _SC_CALL_CACHE.append(_sc_call_build())
    er, ec, hew = _SC_CALL_CACHE[0](posa.reshape(-1), isreal.reshape(-1),
                           rv.reshape(-1), cv.reshape(-1), val.reshape(-1),
                           nnz.reshape(-1))
    return (jnp.stack([er, ec]), hew)


# speedup vs baseline: 5.8732x; 5.8732x over previous
import functools

import jax
import jax.numpy as jnp
from jax import lax
from jax.experimental import pallas as pl
from jax.experimental.pallas import tpu as pltpu
from jax.experimental.pallas import tpu_sc as plsc

N = 4096
K = 20
B = 8
NK = N * K
RB = 128
NB = N // RB
DPAD = 128
EPT = NK // 16
GT = EPT // 128
HPT = 4 * NK // 16

_NOISE01_CACHE = []


def _noise01():
    if not _NOISE01_CACHE:
        _NOISE01_CACHE.append(
            jax.random.uniform(jax.random.key(42), (N, N), dtype=jnp.float32)
            * jnp.float32(0.01))
    return _NOISE01_CACHE[0]


def _tc_body(nv1_ref, nv2_ref, noise_ref,
             posa_ref, isreal_ref, rv_ref, cv_ref, val_ref, nnz_ref,
             ee_ref, carry_ref):
    b = pl.program_id(0)

    @pl.when(b == 0)
    def _init():
        ee_ref[...] = jnp.tanh(2.0 * nv2_ref[...])
        carry_ref[0] = jnp.int32(0)

    de = jnp.tanh(2.0 * nv1_ref[...])
    dot = lax.dot_general(de, ee_ref[...],
                          dimension_numbers=(((1,), (1,)), ((), ())),
                          preferred_element_type=jnp.float32)
    adj = jnp.maximum(jnp.tanh(2.0 * dot), 0.0)
    scores = adj + noise_ref[...]
    col = lax.broadcasted_iota(jnp.int32, (RB, N), 1)
    big = jnp.int32(1 << 30)
    idx_cols = []
    val_cols = []
    for _ in range(K):
        m = jnp.max(scores, axis=1, keepdims=True)
        cand = jnp.where(scores == m, col, big)
        idx_t = jnp.min(cand, axis=1, keepdims=True)
        sel = col == idx_t
        val_t = jnp.sum(jnp.where(sel, adj, 0.0), axis=1, keepdims=True)
        scores = jnp.where(sel, -1.0, scores)
        idx_cols.append(idx_t)
        val_cols.append(val_t)
    idx20 = jnp.concatenate(idx_cols, axis=1)
    val20 = jnp.concatenate(val_cols, axis=1)
    real = val20 > 0.0
    kio = lax.broadcasted_iota(jnp.int32, (RB, K), 1)
    key = jnp.where(real, idx20, N + kio)
    rank = jnp.zeros((RB, K), jnp.int32)
    for j in range(K):
        rank = rank + jnp.where(key[:, j:j + 1] < key, 1, 0)
    cnt = jnp.sum(jnp.where(real, 1, 0), axis=1, keepdims=True)
    rio = lax.broadcasted_iota(jnp.int32, (RB, RB), 0)
    cio = lax.broadcasted_iota(jnp.int32, (RB, RB), 1)
    tril = jnp.where(cio < rio, 1.0, 0.0)
    cstart = lax.dot_general(tril, cnt.astype(jnp.float32),
                             dimension_numbers=(((1,), (0,)), ((), ())),
                             preferred_element_type=jnp.float32)
    carry = carry_ref[0]
    row_start = carry + cstart.astype(jnp.int32)
    carry_ref[0] = carry + jnp.sum(cnt)
    rglob = RB * b + lax.broadcasted_iota(jnp.int32, (RB, 1), 0)
    padstart = K * rglob - row_start
    posa_ref[...] = jnp.where(real, row_start + rank, padstart + rank - cnt)
    isreal_ref[...] = jnp.where(real, 1, 0)
    rv_ref[...] = jnp.where(real, rglob, 0)
    cv_ref[...] = jnp.where(real, idx20, 0)
    val_ref[...] = val20
    nnz_ref[...] = jnp.full((8, 128), carry_ref[0], jnp.int32)


_tc_call = pl.pallas_call(
    _tc_body,
    grid=(NB,),
    in_specs=[
        pl.BlockSpec((RB, DPAD), lambda b: (b, 0)),
        pl.BlockSpec((N, DPAD), lambda b: (0, 0)),
        pl.BlockSpec((RB, N), lambda b: (b, 0)),
    ],
    out_specs=[
        pl.BlockSpec((RB, K), lambda b: (b, 0)),
        pl.BlockSpec((RB, K), lambda b: (b, 0)),
        pl.BlockSpec((RB, K), lambda b: (b, 0)),
        pl.BlockSpec((RB, K), lambda b: (b, 0)),
        pl.BlockSpec((RB, K), lambda b: (b, 0)),
        pl.BlockSpec((8, 128), lambda b: (0, 0)),
    ],
    out_shape=[
        jax.ShapeDtypeStruct((N, K), jnp.int32),
        jax.ShapeDtypeStruct((N, K), jnp.int32),
        jax.ShapeDtypeStruct((N, K), jnp.int32),
        jax.ShapeDtypeStruct((N, K), jnp.int32),
        jax.ShapeDtypeStruct((N, K), jnp.float32),
        jax.ShapeDtypeStruct((8, 128), jnp.int32),
    ],
    scratch_shapes=[
        pltpu.VMEM((N, DPAD), jnp.float32),
        pltpu.SMEM((1,), jnp.int32),
    ],
)


def _sc_body(posa_hbm, isreal_hbm, rv_hbm, cv_hbm, val_hbm, nnz_hbm,
             er_hbm, ec_hbm, hew_hbm,
             posa_v, isreal_v, rv_v, cv_v, val_v, nnz_v,
             idxa_v, idxg_v, buf_v, hbuf_v,
             er0_sh, ec0_sh, hew0_sh, sem):
    c = lax.axis_index("c")
    s = lax.axis_index("s")
    base = s * EPT
    with jax.named_scope("sc_load"):
        pltpu.sync_copy(posa_hbm.at[pl.ds(base, EPT)], posa_v)
        pltpu.sync_copy(isreal_hbm.at[pl.ds(base, EPT)], isreal_v)
        pltpu.sync_copy(rv_hbm.at[pl.ds(base, EPT)], rv_v)
        pltpu.sync_copy(cv_hbm.at[pl.ds(base, EPT)], cv_v)
        pltpu.sync_copy(val_hbm.at[pl.ds(base, EPT)], val_v)
        pltpu.sync_copy(nnz_hbm.at[pl.ds(0, 16)], nnz_v)
    nnz = nnz_v[...]

    with jax.named_scope("sc_build"):
        def build_body(g, _):
            for v in range(8):
                off = g * 128 + v * 16
                pos = posa_v[pl.ds(off, 16)]
                pad = 1 - isreal_v[pl.ds(off, 16)]
                idxa_v[g, pl.ds(v * 16, 16)] = pos + pad * nnz
            pltpu.async_copy(rv_v.at[pl.ds(g * 128, 128)],
                             er0_sh.at[idxa_v.at[g]], sem)
            pltpu.async_copy(cv_v.at[pl.ds(g * 128, 128)],
                             ec0_sh.at[idxa_v.at[g]], sem)
            pltpu.async_copy(val_v.at[pl.ds(g * 128, 128)],
                             hew0_sh.at[idxa_v.at[g]], sem)
            return 0

        lax.fori_loop(0, GT, build_body, 0)
        pltpu.make_async_copy(posa_hbm.at[pl.ds(0, EPT)], posa_v, sem).wait()
        pltpu.make_async_copy(posa_hbm.at[pl.ds(0, EPT)], rv_v, sem).wait()
        pltpu.make_async_copy(posa_hbm.at[pl.ds(0, EPT)], cv_v, sem).wait()
    plsc.subcore_barrier()

    with jax.named_scope("sc_emit_e"):
        for ib in range(4):
            i = 4 * c + ib
            for src_sh, dst_hbm in ((er0_sh, er_hbm), (ec0_sh, ec_hbm)):
                pltpu.sync_copy(src_sh.at[pl.ds(base, EPT)], buf_v)

                def add_body(m, _):
                    for v in range(8):
                        off = m * 128 + v * 16
                        buf_v[pl.ds(off, 16)] = buf_v[pl.ds(off, 16)] + i * N
                    return 0

                lax.fori_loop(0, GT, add_body, 0)
                pltpu.sync_copy(buf_v, dst_hbm.at[pl.ds(i * NK + base, EPT)])

    with jax.named_scope("sc_emit_h"):
        j0 = c * (4 * NK) + s * HPT
        lane = lax.iota(jnp.int32, 16)

        def hew_body(g, _):
            for v in range(8):
                off = g * 128 + v * 16
                j = j0 + off + lane
                bi = jnp.zeros((16,), jnp.int32)
                for m in range(1, 8):
                    bi = bi + jnp.where(j >= m * nnz, 1, 0)
                src = jnp.minimum(j - bi * nnz, NK - 1)
                idxg_v[g, pl.ds(v * 16, 16)] = src
            pltpu.async_copy(hew0_sh.at[idxg_v.at[g]],
                             hbuf_v.at[pl.ds(g * 128, 128)], sem)
            return 0

        lax.fori_loop(0, HPT // 128, hew_body, 0)
        pltpu.make_async_copy(hew_hbm.at[pl.ds(0, HPT)], hbuf_v, sem).wait()
        pltpu.sync_copy(hbuf_v, hew_hbm.at[pl.ds(j0, HPT)])


_SC_CALL_CACHE = []


def _sc_call_build():
    return functools.partial(
        pl.kernel,
        mesh=plsc.VectorSubcoreMesh(core_axis_name="c", subcore_axis_name="s"),
        out_type=[
            jax.ShapeDtypeStruct((B * NK,), jnp.int32),
            jax.ShapeDtypeStruct((B * NK,), jnp.int32),
            jax.ShapeDtypeStruct((B * NK,), jnp.float32),
        ],
        scratch_types=[
            pltpu.VMEM((EPT,), jnp.int32),
            pltpu.VMEM((EPT,), jnp.int32),
            pltpu.VMEM((EPT,), jnp.int32),
            pltpu.VMEM((EPT,), jnp.int32),
            pltpu.VMEM((EPT,), jnp.float32),
            pltpu.VMEM((16,), jnp.int32),
            pltpu.VMEM((GT, 128), jnp.int32),
            pltpu.VMEM((HPT // 128, 128), jnp.int32),
            pltpu.VMEM((EPT,), jnp.int32),
            pltpu.VMEM((HPT,), jnp.float32),
            pltpu.VMEM_SHARED((NK,), jnp.int32),
            pltpu.VMEM_SHARED((NK,), jnp.int32),
            pltpu.VMEM_SHARED((NK,), jnp.float32),
            pltpu.SemaphoreType.DMA,
        ],
    )(_sc_body)


def kernel(x, nodevec1, nodevec2):
    del x
    nv1 = jnp.pad(nodevec1, ((0, 0), (0, DPAD - nodevec1.shape[1])))
    nv2 = jnp.pad(nodevec2, ((0, 0), (0, DPAD - nodevec2.shape[1])))
    posa, isreal, rv, cv, val, nnz = _tc_call(nv1, nv2, _noise01())
    if not _SC_CALL_CACHE:
        _SC_CALL_CACHE.append(_sc_call_build())
    er, ec, hew = _SC_CALL_CACHE[0](posa.reshape(-1), isreal.reshape(-1),
                           rv.reshape(-1), cv.reshape(-1), val.reshape(-1),
                           nnz.reshape(-1))
    return (jnp.stack([er, ec]), hew)


# speedup vs baseline: 6.4268x; 1.0943x over previous
import functools

import jax
import jax.numpy as jnp
from jax import lax
from jax.experimental import pallas as pl
from jax.experimental.pallas import tpu as pltpu
from jax.experimental.pallas import tpu_sc as plsc

N = 4096
K = 20
B = 8
NK = N * K
RB = 128
NB = N // RB
DPAD = 128
EPT = NK // 16
GT = EPT // 128
HPT = 4 * NK // 16

_NOISE01_CACHE = []


def _noise01():
    if not _NOISE01_CACHE:
        _NOISE01_CACHE.append(
            jax.random.uniform(jax.random.key(42), (N, N), dtype=jnp.float32)
            * jnp.float32(0.01))
    return _NOISE01_CACHE[0]


def _tc_body(nv1_ref, nv2_ref, noise_ref,
             posa_ref, isreal_ref, rv_ref, cv_ref, val_ref, nnz_ref,
             ee_ref, carry_ref):
    b = pl.program_id(0)

    @pl.when(b == 0)
    def _init():
        ee_ref[...] = jnp.tanh(2.0 * nv2_ref[...])
        carry_ref[0] = jnp.int32(0)

    de = jnp.tanh(2.0 * nv1_ref[...])
    dot = lax.dot_general(de, ee_ref[...],
                          dimension_numbers=(((1,), (1,)), ((), ())),
                          preferred_element_type=jnp.float32)
    adj = jnp.maximum(jnp.tanh(2.0 * dot), 0.0)
    scores = adj + noise_ref[...]
    colr = jnp.float32(N - 1) - lax.broadcasted_iota(
        jnp.int32, (RB, N), 1).astype(jnp.float32)
    negf = jnp.float32(-1e9)
    idx_cols = []
    val_cols = []
    for _ in range(K):
        m = jnp.max(scores, axis=1, keepdims=True)
        colsel = jnp.where(scores == m, colr, negf)
        m2 = jnp.max(colsel, axis=1, keepdims=True)
        sel = colsel == m2
        val_t = jnp.sum(jnp.where(sel, adj, 0.0), axis=1, keepdims=True)
        scores = jnp.where(sel, -1.0, scores)
        idx_cols.append(jnp.int32(N - 1) - m2.astype(jnp.int32))
        val_cols.append(val_t)
    idx20 = jnp.concatenate(idx_cols, axis=1)
    val20 = jnp.concatenate(val_cols, axis=1)
    real = val20 > 0.0
    kio = lax.broadcasted_iota(jnp.int32, (RB, K), 1)
    key = jnp.where(real, idx20, N + kio)
    rank = jnp.zeros((RB, K), jnp.int32)
    for j in range(K):
        rank = rank + jnp.where(key[:, j:j + 1] < key, 1, 0)
    cnt = jnp.sum(jnp.where(real, 1, 0), axis=1, keepdims=True)
    rio = lax.broadcasted_iota(jnp.int32, (RB, RB), 0)
    cio = lax.broadcasted_iota(jnp.int32, (RB, RB), 1)
    tril = jnp.where(cio < rio, 1.0, 0.0)
    cstart = lax.dot_general(tril, cnt.astype(jnp.float32),
                             dimension_numbers=(((1,), (0,)), ((), ())),
                             preferred_element_type=jnp.float32)
    carry = carry_ref[0]
    row_start = carry + cstart.astype(jnp.int32)
    carry_ref[0] = carry + jnp.sum(cnt)
    rglob = RB * b + lax.broadcasted_iota(jnp.int32, (RB, 1), 0)
    padstart = K * rglob - row_start
    posa_ref[...] = jnp.where(real, row_start + rank, padstart + rank - cnt)
    isreal_ref[...] = jnp.where(real, 1, 0)
    rv_ref[...] = jnp.where(real, rglob, 0)
    cv_ref[...] = jnp.where(real, idx20, 0)
    val_ref[...] = val20
    nnz_ref[...] = jnp.full((8, 128), carry_ref[0], jnp.int32)


_tc_call = pl.pallas_call(
    _tc_body,
    grid=(NB,),
    in_specs=[
        pl.BlockSpec((RB, DPAD), lambda b: (b, 0)),
        pl.BlockSpec((N, DPAD), lambda b: (0, 0)),
        pl.BlockSpec((RB, N), lambda b: (b, 0)),
    ],
    out_specs=[
        pl.BlockSpec((RB, K), lambda b: (b, 0)),
        pl.BlockSpec((RB, K), lambda b: (b, 0)),
        pl.BlockSpec((RB, K), lambda b: (b, 0)),
        pl.BlockSpec((RB, K), lambda b: (b, 0)),
        pl.BlockSpec((RB, K), lambda b: (b, 0)),
        pl.BlockSpec((8, 128), lambda b: (0, 0)),
    ],
    out_shape=[
        jax.ShapeDtypeStruct((N, K), jnp.int32),
        jax.ShapeDtypeStruct((N, K), jnp.int32),
        jax.ShapeDtypeStruct((N, K), jnp.int32),
        jax.ShapeDtypeStruct((N, K), jnp.int32),
        jax.ShapeDtypeStruct((N, K), jnp.float32),
        jax.ShapeDtypeStruct((8, 128), jnp.int32),
    ],
    scratch_shapes=[
        pltpu.VMEM((N, DPAD), jnp.float32),
        pltpu.SMEM((1,), jnp.int32),
    ],
)


def _sc_body(posa_hbm, isreal_hbm, rv_hbm, cv_hbm, val_hbm, nnz_hbm,
             er_hbm, ec_hbm, hew_hbm,
             posa_v, isreal_v, rv_v, cv_v, val_v, nnz_v,
             idxa_v, idxg_v, buf_v, hbuf_v,
             er0_sh, ec0_sh, hew0_sh, sem):
    c = lax.axis_index("c")
    s = lax.axis_index("s")
    base = s * EPT
    with jax.named_scope("sc_load"):
        pltpu.sync_copy(posa_hbm.at[pl.ds(base, EPT)], posa_v)
        pltpu.sync_copy(isreal_hbm.at[pl.ds(base, EPT)], isreal_v)
        pltpu.sync_copy(rv_hbm.at[pl.ds(base, EPT)], rv_v)
        pltpu.sync_copy(cv_hbm.at[pl.ds(base, EPT)], cv_v)
        pltpu.sync_copy(val_hbm.at[pl.ds(base, EPT)], val_v)
        pltpu.sync_copy(nnz_hbm.at[pl.ds(0, 16)], nnz_v)
    nnz = nnz_v[...]

    with jax.named_scope("sc_build"):
        def build_body(g, _):
            for v in range(8):
                off = g * 128 + v * 16
                pos = posa_v[pl.ds(off, 16)]
                pad = 1 - isreal_v[pl.ds(off, 16)]
                idxa_v[g, pl.ds(v * 16, 16)] = pos + pad * nnz
            pltpu.async_copy(rv_v.at[pl.ds(g * 128, 128)],
                             er0_sh.at[idxa_v.at[g]], sem)
            pltpu.async_copy(cv_v.at[pl.ds(g * 128, 128)],
                             ec0_sh.at[idxa_v.at[g]], sem)
            pltpu.async_copy(val_v.at[pl.ds(g * 128, 128)],
                             hew0_sh.at[idxa_v.at[g]], sem)
            return 0

        lax.fori_loop(0, GT, build_body, 0)
        pltpu.make_async_copy(posa_hbm.at[pl.ds(0, EPT)], posa_v, sem).wait()
        pltpu.make_async_copy(posa_hbm.at[pl.ds(0, EPT)], rv_v, sem).wait()
        pltpu.make_async_copy(posa_hbm.at[pl.ds(0, EPT)], cv_v, sem).wait()
    plsc.subcore_barrier()

    with jax.named_scope("sc_emit_e"):
        for ib in range(4):
            i = 4 * c + ib
            for src_sh, dst_hbm in ((er0_sh, er_hbm), (ec0_sh, ec_hbm)):
                pltpu.sync_copy(src_sh.at[pl.ds(base, EPT)], buf_v)

                def add_body(m, _):
                    for v in range(8):
                        off = m * 128 + v * 16
                        buf_v[pl.ds(off, 16)] = buf_v[pl.ds(off, 16)] + i * N
                    return 0

                lax.fori_loop(0, GT, add_body, 0)
                pltpu.sync_copy(buf_v, dst_hbm.at[pl.ds(i * NK + base, EPT)])

    with jax.named_scope("sc_emit_h"):
        j0 = c * (4 * NK) + s * HPT
        lane = lax.iota(jnp.int32, 16)

        def hew_body(g, _):
            for v in range(8):
                off = g * 128 + v * 16
                j = j0 + off + lane
                bi = jnp.zeros((16,), jnp.int32)
                for m in range(1, 8):
                    bi = bi + jnp.where(j >= m * nnz, 1, 0)
                src = jnp.minimum(j - bi * nnz, NK - 1)
                idxg_v[g, pl.ds(v * 16, 16)] = src
            pltpu.async_copy(hew0_sh.at[idxg_v.at[g]],
                             hbuf_v.at[pl.ds(g * 128, 128)], sem)
            return 0

        lax.fori_loop(0, HPT // 128, hew_body, 0)
        pltpu.make_async_copy(hew_hbm.at[pl.ds(0, HPT)], hbuf_v, sem).wait()
        pltpu.sync_copy(hbuf_v, hew_hbm.at[pl.ds(j0, HPT)])


_SC_CALL_CACHE = []


def _sc_call_build():
    return functools.partial(
        pl.kernel,
        mesh=plsc.VectorSubcoreMesh(core_axis_name="c", subcore_axis_name="s"),
        out_type=[
            jax.ShapeDtypeStruct((B * NK,), jnp.int32),
            jax.ShapeDtypeStruct((B * NK,), jnp.int32),
            jax.ShapeDtypeStruct((B * NK,), jnp.float32),
        ],
        scratch_types=[
            pltpu.VMEM((EPT,), jnp.int32),
            pltpu.VMEM((EPT,), jnp.int32),
            pltpu.VMEM((EPT,), jnp.int32),
            pltpu.VMEM((EPT,), jnp.int32),
            pltpu.VMEM((EPT,), jnp.float32),
            pltpu.VMEM((16,), jnp.int32),
            pltpu.VMEM((GT, 128), jnp.int32),
            pltpu.VMEM((HPT // 128, 128), jnp.int32),
            pltpu.VMEM((EPT,), jnp.int32),
            pltpu.VMEM((HPT,), jnp.float32),
            pltpu.VMEM_SHARED((NK,), jnp.int32),
            pltpu.VMEM_SHARED((NK,), jnp.int32),
            pltpu.VMEM_SHARED((NK,), jnp.float32),
            pltpu.SemaphoreType.DMA,
        ],
    )(_sc_body)


def kernel(x, nodevec1, nodevec2):
    del x
    nv1 = jnp.pad(nodevec1, ((0, 0), (0, DPAD - nodevec1.shape[1])))
    nv2 = jnp.pad(nodevec2, ((0, 0), (0, DPAD - nodevec2.shape[1])))
    posa, isreal, rv, cv, val, nnz = _tc_call(nv1, nv2, _noise01())
    if not _SC_CALL_CACHE:
        _SC_CALL_CACHE.append(_sc_call_build())
    er, ec, hew = _SC_CALL_CACHE[0](posa.reshape(-1), isreal.reshape(-1),
                           rv.reshape(-1), cv.reshape(-1), val.reshape(-1),
                           nnz.reshape(-1))
    return (jnp.stack([er, ec]), hew)
